# baseline (speedup 1.0000x reference)
import functools

import jax
import jax.numpy as jnp
from jax import lax
from jax.experimental import pallas as pl
from jax.experimental.pallas import tpu as pltpu
from jax.experimental.pallas import tpu_sc as plsc

_DIM = 32
_CHUNK = 128


def _make_sc_kernel(batch):
    info = plsc.get_sparse_core_info()
    nc, ns, lanes = info.num_cores, info.num_subcores, info.num_lanes
    nw = nc * ns
    bpw = batch // nw
    nch = bpw // _CHUNK

    mesh = plsc.VectorSubcoreMesh(core_axis_name="c", subcore_axis_name="s")

    @functools.partial(
        pl.kernel,
        mesh=mesh,
        compiler_params=pltpu.CompilerParams(
            needs_layout_passes=False, use_tc_tiling_on_sc=False),
        out_type=jax.ShapeDtypeStruct((batch,), jnp.float32),
        scratch_types=[
            pltpu.VMEM((bpw,), jnp.int32),
            pltpu.VMEM((bpw,), jnp.int32),
            pltpu.VMEM((bpw, _DIM), jnp.float32),
            pltpu.VMEM((bpw, _DIM), jnp.float32),
            pltpu.VMEM((bpw,), jnp.float32),
            pltpu.VMEM((lanes,), jnp.float32),
            pltpu.SemaphoreType.DMA,
        ],
    )
    def k(user_hbm, item_hbm, uemb_hbm, iemb_hbm, bias_hbm, out_hbm,
          uidx_v, iidx_v, urows_v, irows_v, out_v, bias_v, sem):
        wid = lax.axis_index("s") * nc + lax.axis_index("c")
        base = wid * bpw

        pltpu.sync_copy(user_hbm.at[pl.ds(base, bpw)], uidx_v)
        pltpu.sync_copy(item_hbm.at[pl.ds(base, bpw)], iidx_v)
        pltpu.sync_copy(bias_hbm, bias_v)

        copies = []
        for c in range(nch):
            sl = pl.ds(c * _CHUNK, _CHUNK)
            copies.append(
                pltpu.async_copy(uemb_hbm.at[uidx_v.at[sl]], urows_v.at[sl], sem))
            copies.append(
                pltpu.async_copy(iemb_hbm.at[iidx_v.at[sl]], irows_v.at[sl], sem))
        for cp in copies:
            cp.wait()

        lane_iota = lax.iota(jnp.int32, lanes)
        bias_vec = bias_v[...]

        def body(g, carry):
            rows16 = g * lanes + lane_iota
            acc = jnp.zeros((lanes,), jnp.float32)
            for d in range(_DIM):
                col = jnp.full((lanes,), d, jnp.int32)
                uv = plsc.load_gather(urows_v, [rows16, col])
                iv = plsc.load_gather(irows_v, [rows16, col])
                acc = acc + uv * iv
            z = acc + bias_vec
            res = 1.0 / (1.0 + jnp.exp(-z))
            out_v[pl.ds(g * lanes, lanes)] = res
            return carry

        lax.fori_loop(0, bpw // lanes, body, 0)

        pltpu.sync_copy(out_v, out_hbm.at[pl.ds(base, bpw)])

    return k


def kernel(user, item, user_emb, item_emb, bias):
    batch = user.shape[0]
    lanes = plsc.get_sparse_core_info().num_lanes
    user = user.astype(jnp.int32)
    item = item.astype(jnp.int32)
    bias16 = jnp.broadcast_to(bias.astype(jnp.float32), (lanes,))
    k = _make_sc_kernel(batch)
    return k(user, item, user_emb, item_emb, bias16)


# speedup vs baseline: 3.5822x; 3.5822x over previous
import functools

import jax
import jax.numpy as jnp
from jax import lax
from jax.experimental import pallas as pl
from jax.experimental.pallas import tpu as pltpu
from jax.experimental.pallas import tpu_sc as plsc

_DIM = 32
_BLK = 128
_GRP = 8


def _make_sc_kernel(batch):
    info = plsc.get_sparse_core_info()
    nc, ns, lanes = info.num_cores, info.num_subcores, info.num_lanes
    nw = nc * ns
    bpw = batch // nw

    mesh = plsc.VectorSubcoreMesh(core_axis_name="c", subcore_axis_name="s")

    @functools.partial(
        pl.kernel,
        mesh=mesh,
        compiler_params=pltpu.CompilerParams(
            needs_layout_passes=False, use_tc_tiling_on_sc=True),
        out_type=jax.ShapeDtypeStruct((batch,), jnp.float32),
        scratch_types=[
            pltpu.VMEM((bpw,), jnp.int32),
            pltpu.VMEM((bpw,), jnp.int32),
            pltpu.VMEM((_DIM, _GRP * _BLK), jnp.float32),
            pltpu.VMEM((_DIM, _GRP * _BLK), jnp.float32),
            pltpu.VMEM((bpw,), jnp.float32),
            pltpu.VMEM((lanes,), jnp.float32),
            pltpu.SemaphoreType.DMA,
        ],
    )
    def k(user_hbm, item_hbm, ut_hbm, it_hbm, bias_hbm, out_hbm,
          uidx_v, iidx_v, ubuf, ibuf, out_v, bias_v, sem):
        wid = lax.axis_index("s") * nc + lax.axis_index("c")
        base = wid * bpw

        pltpu.sync_copy(user_hbm.at[pl.ds(base, bpw)], uidx_v)
        pltpu.sync_copy(item_hbm.at[pl.ds(base, bpw)], iidx_v)
        pltpu.sync_copy(bias_hbm, bias_v)

        lane_iota = lax.iota(jnp.int32, lanes)
        d_lo = lane_iota
        d_hi = lane_iota + lanes
        bias_vec = bias_v[...]

        def body(g, carry):
            uchunk = uidx_v[pl.ds(g * lanes, lanes)]
            ichunk = iidx_v[pl.ds(g * lanes, lanes)]
            acc = jnp.zeros((lanes,), jnp.float32)
            for h in range(lanes // _GRP):
                copies = []
                for kk in range(_GRP):
                    ru = uchunk[h * _GRP + kk]
                    ri = ichunk[h * _GRP + kk]
                    su = pl.multiple_of((ru >> 7) * _BLK, _BLK)
                    si = pl.multiple_of((ri >> 7) * _BLK, _BLK)
                    dst = pl.ds(kk * _BLK, _BLK)
                    copies.append(pltpu.async_copy(
                        ut_hbm.at[:, pl.ds(su, _BLK)], ubuf.at[:, dst], sem))
                    copies.append(pltpu.async_copy(
                        it_hbm.at[:, pl.ds(si, _BLK)], ibuf.at[:, dst], sem))
                for cp in copies:
                    cp.wait()
                for kk in range(_GRP):
                    cu = (uchunk[h * _GRP + kk] & (_BLK - 1)) + kk * _BLK
                    ci = (ichunk[h * _GRP + kk] & (_BLK - 1)) + kk * _BLK
                    cuv = jnp.full((lanes,), cu, jnp.int32)
                    civ = jnp.full((lanes,), ci, jnp.int32)
                    u0 = plsc.load_gather(ubuf, [d_lo, cuv])
                    u1 = plsc.load_gather(ubuf, [d_hi, cuv])
                    i0 = plsc.load_gather(ibuf, [d_lo, civ])
                    i1 = plsc.load_gather(ibuf, [d_hi, civ])
                    dot = jnp.sum(u0 * i0 + u1 * i1)
                    pos = h * _GRP + kk
                    acc = jnp.where(lane_iota == pos,
                                    jnp.full((lanes,), dot, jnp.float32), acc)
            z = acc + bias_vec
            out_v[pl.ds(g * lanes, lanes)] = 1.0 / (1.0 + jnp.exp(-z))
            return carry

        lax.fori_loop(0, bpw // lanes, body, 0)

        pltpu.sync_copy(out_v, out_hbm.at[pl.ds(base, bpw)])

    return k


def kernel(user, item, user_emb, item_emb, bias):
    batch = user.shape[0]
    lanes = plsc.get_sparse_core_info().num_lanes
    user = user.astype(jnp.int32)
    item = item.astype(jnp.int32)
    bias16 = jnp.broadcast_to(bias.astype(jnp.float32), (lanes,))
    k = _make_sc_kernel(batch)
    return k(user, item, user_emb.T, item_emb.T, bias16)
